# baseline (speedup 1.0000x reference)
import functools

import jax
import jax.numpy as jnp
from jax import lax
from jax.experimental import pallas as pl
from jax.experimental.pallas import tpu as pltpu
from jax.experimental.pallas import tpu_sc as plsc
from jax.experimental.layout import Format, Layout

NB = 4


@functools.cache
def _build(B0, S, V, D):
    info = plsc.get_sparse_core_info()
    NC, NS = info.num_cores, info.num_subcores
    NW = NC * NS
    L = 16
    assert B0 == 128 * NW and D % 8 == 0 and S % NB == 0
    DT = D // 8
    mesh = plsc.VectorSubcoreMesh(core_axis_name="c", subcore_axis_name="s")

    @functools.partial(
        pl.kernel,
        mesh=mesh,
        compiler_params=pltpu.CompilerParams(use_tc_tiling_on_sc=False,
                                             needs_layout_passes=False),
        out_type=jax.ShapeDtypeStruct((S, DT, NW, 8, 128), jnp.float32),
        scratch_types=[
            pltpu.VMEM((S, 128), jnp.int32),
            pltpu.VMEM((NB, 128, D), jnp.float32),
            pltpu.VMEM((NB, DT, 8, 128), jnp.float32),
            pltpu.SemaphoreType.DMA((NB,)),
            pltpu.SemaphoreType.DMA((NB,)),
        ],
    )
    def emb_kernel(xT_hbm, table_hbm, out_hbm, idxT, rows_v, obuf, gsem,
                   ssem):
        wid = lax.axis_index("s") * NC + lax.axis_index("c")
        pltpu.sync_copy(xT_hbm.at[:, pl.ds(wid * 128, 128)], idxT)

        lane = lax.iota(jnp.int32, L)
        row_base = None

        def fire_gather(s, i):
            pltpu.async_copy(table_hbm.at[idxT.at[s]], rows_v.at[i],
                             gsem.at[i])

        def wait_gather(s, i):
            pltpu.make_async_copy(table_hbm.at[idxT.at[s]], rows_v.at[i],
                                  gsem.at[i]).wait()

        def fire_store(s, i):
            pltpu.async_copy(obuf.at[i], out_hbm.at[s, pl.ds(0, DT), wid],
                             ssem.at[i])

        def wait_store(s, i):
            pltpu.make_async_copy(obuf.at[i],
                                  out_hbm.at[s, pl.ds(0, DT), wid],
                                  ssem.at[i]).wait()

        for i in range(NB):
            fire_gather(i, i)

        def body(t, carry):
            for i in range(NB):
                s = t * NB + i
                wait_gather(s, i)

                @pl.when(s >= NB)
                def _():
                    wait_store(s - NB, i)

                rows = rows_v.at[i]
                ob = obuf.at[i]

                def obody(o, c):
                    rotv = jnp.bitwise_and(lane + o, L - 1)
                    dtv = jnp.right_shift(rotv, 3)
                    dsv = jnp.bitwise_and(rotv, 7)
                    for j in range(D // L):
                        colj = rotv + L * j
                        dtj = dtv + 2 * j
                        for k in range(128 // L):
                            blk = lane + L * k
                            v = plsc.load_gather(rows, [blk, colj])
                            plsc.store_scatter(ob, [dtj, dsv, blk], v)
                    return c

                lax.fori_loop(0, L, obody, 0)
                fire_store(s, i)

                @pl.when(s + NB < S)
                def _():
                    fire_gather(s + NB, i)

            return carry

        lax.fori_loop(0, S // NB, body, 0)

        for i in range(NB):
            wait_store(S - NB + i, i)

    return emb_kernel


def kernel(x, table):
    B0, S = x.shape
    V, D = table.shape
    tab2 = jnp.stack([table, table], axis=1).reshape(2 * V, D)
    xT = (x.astype(jnp.int32) * 2).T
    out6 = _build(B0, S, 2 * V, D)(xT, tab2)
    return out6.transpose(2, 4, 0, 1, 3).reshape(B0, S, D)


# speedup vs baseline: 1.8261x; 1.8261x over previous
import functools

import jax
import jax.numpy as jnp
from jax import lax
from jax.experimental import pallas as pl
from jax.experimental.pallas import tpu as pltpu
from jax.experimental.pallas import tpu_sc as plsc
from jax.experimental.layout import Format, Layout

NB = 4


@functools.cache
def _build(B0, S, V, D):
    info = plsc.get_sparse_core_info()
    NC, NS = info.num_cores, info.num_subcores
    NW = NC * NS
    L = 16
    assert B0 == 128 * NW and D % 8 == 0 and S % NB == 0
    DT = D // 8
    mesh = plsc.VectorSubcoreMesh(core_axis_name="c", subcore_axis_name="s")

    @functools.partial(
        pl.kernel,
        mesh=mesh,
        compiler_params=pltpu.CompilerParams(use_tc_tiling_on_sc=False,
                                             needs_layout_passes=False),
        out_type=jax.ShapeDtypeStruct((S, DT, NW, 8, 128), jnp.float32),
        scratch_types=[
            pltpu.VMEM((S, 128), jnp.int32),
            pltpu.VMEM((NB, 128, D), jnp.float32),
            pltpu.VMEM((NB, DT, 8, 128), jnp.float32),
            pltpu.SemaphoreType.DMA((NB,)),
            pltpu.SemaphoreType.DMA((NB,)),
        ],
    )
    def emb_kernel(xT_hbm, table_hbm, out_hbm, idxT, rows_v, obuf, gsem,
                   ssem):
        wid = lax.axis_index("s") * NC + lax.axis_index("c")
        pltpu.sync_copy(xT_hbm.at[:, pl.ds(wid * 128, 128)], idxT)

        lane = lax.iota(jnp.int32, L)

        def fire_gather(s, i):
            pltpu.async_copy(table_hbm.at[idxT.at[s]], rows_v.at[i],
                             gsem.at[i])

        def wait_gather(s, i):
            pltpu.make_async_copy(table_hbm.at[idxT.at[s]], rows_v.at[i],
                                  gsem.at[i]).wait()

        def fire_store(s, i):
            pltpu.async_copy(obuf.at[i], out_hbm.at[s, pl.ds(0, DT), wid],
                             ssem.at[i])

        def wait_store(s, i):
            pltpu.make_async_copy(obuf.at[i],
                                  out_hbm.at[s, pl.ds(0, DT), wid],
                                  ssem.at[i]).wait()

        for i in range(NB):
            fire_gather(i, i)

        def body(t, carry):
            for i in range(NB):
                s = t * NB + i
                wait_gather(s, i)

                @pl.when(s >= NB)
                def _():
                    wait_store(s - NB, i)

                rows = rows_v.at[i]
                ob = obuf.at[i]

                def obody(o, c):
                    rotv = jnp.bitwise_and(lane + o, L - 1)
                    dtv = jnp.right_shift(rotv, 3)
                    dsv = jnp.bitwise_and(rotv, 7)
                    for j in range(D // L):
                        colj = rotv + L * j
                        dtj = dtv + 2 * j
                        for k in range(128 // L):
                            blk = lane + L * k
                            v = plsc.load_gather(rows, [blk, colj])
                            plsc.store_scatter(ob, [dtj, dsv, blk], v)
                    return c

                lax.fori_loop(0, L, obody, 0)
                fire_store(s, i)

                @pl.when(s + NB < S)
                def _():
                    fire_gather(s + NB, i)

            return carry

        lax.fori_loop(0, S // NB, body, 0)

        for i in range(NB):
            wait_store(S - NB + i, i)

    return emb_kernel


def kernel(x, table):
    B0, S = x.shape
    V, D = table.shape
    tab_l = jax.device_put(
        table,
        Format(Layout(major_to_minor=(0, 1), tiling=((8,),)),
               jax.sharding.SingleDeviceSharding(jax.devices()[0])))
    xT = x.astype(jnp.int32).T
    out6 = _build(B0, S, V, D)(xT, tab_l)
    return out6.transpose(2, 4, 0, 1, 3).reshape(B0, S, D)
